# Initial kernel scaffold; baseline (speedup 1.0000x reference)
import jax
import jax.numpy as jnp
from jax.experimental import pallas as pl


def kernel(x, W_g, W_gate, W_up, W_down, W_gate_s, W_up_s, W_down_s):
    raise NotImplementedError("write your pallas kernel here")


# speedup vs baseline: 1.5268x; 1.5268x over previous
import functools

import jax
import jax.numpy as jnp
from jax import lax
from jax.experimental import pallas as pl
from jax.experimental.pallas import tpu as pltpu
from jax.experimental.pallas import tpu_sc as plsc

B, S, D = 1, 2048, 2048
E, K, DE = 8, 2, 1024
N_SHARED = 2
DS = DE * N_SHARED

TM = 256
NB_R = (S * K) // TM + E
P_R = NB_R * TM
NB = NB_R + N_SHARED * (S // TM)
P = NB * TM

_SC_CORES = 2
_SC_SUBCORES = 16
_NW = _SC_CORES * _SC_SUBCORES


def _gather_rows(table, idx, chunk=32):
    n_rows, d = table.shape
    b = idx.shape[0]
    assert b % (_NW * chunk) == 0 and b % (8 * _NW) == 0
    b_per_w = b // _NW
    mesh = plsc.VectorSubcoreMesh(core_axis_name="c", subcore_axis_name="s")

    @functools.partial(
        pl.kernel,
        mesh=mesh,
        out_type=jax.ShapeDtypeStruct((b, d), table.dtype),
        scratch_types=[
            pltpu.VMEM((b_per_w,), jnp.int32),
            pltpu.VMEM((chunk, d), table.dtype),
            pltpu.SemaphoreType.DMA,
        ],
    )
    def k(table_hbm, idx_hbm, out_hbm, idx_v, rows_v, sem):
        wid = lax.axis_index("s") * _SC_CORES + lax.axis_index("c")
        base = wid * b_per_w
        pltpu.sync_copy(idx_hbm.at[pl.ds(base, b_per_w)], idx_v)
        for c in range(b_per_w // chunk):
            pltpu.async_copy(
                table_hbm.at[idx_v.at[pl.ds(c * chunk, chunk)]], rows_v, sem
            ).wait()
            pltpu.sync_copy(rows_v, out_hbm.at[pl.ds(base + c * chunk, chunk)])

    return k(table, idx)


def _grouped_ffn_body(gid_ref, xt_ref, wg_ref, wu_ref, wd_ref, w_ref, o_ref):
    x = xt_ref[...].astype(jnp.bfloat16)
    g = jnp.dot(x, wg_ref[0], preferred_element_type=jnp.float32)
    u = jnp.dot(x, wu_ref[0], preferred_element_type=jnp.float32)
    h = (g * jax.nn.sigmoid(g) * u).astype(jnp.bfloat16)
    o = jnp.dot(h, wd_ref[0], preferred_element_type=jnp.float32)
    o_ref[...] = o * w_ref[...]


def _grouped_ffn(gid, xt, wg_all, wu_all, wd_all, w2d):
    grid_spec = pltpu.PrefetchScalarGridSpec(
        num_scalar_prefetch=1,
        grid=(NB,),
        in_specs=[
            pl.BlockSpec((TM, D), lambda i, g: (i, 0)),
            pl.BlockSpec((1, D, DE), lambda i, g: (g[i], 0, 0)),
            pl.BlockSpec((1, D, DE), lambda i, g: (g[i], 0, 0)),
            pl.BlockSpec((1, DE, D), lambda i, g: (g[i], 0, 0)),
            pl.BlockSpec((TM, 1), lambda i, g: (i, 0)),
        ],
        out_specs=pl.BlockSpec((TM, D), lambda i, g: (i, 0)),
    )
    return pl.pallas_call(
        _grouped_ffn_body,
        grid_spec=grid_spec,
        out_shape=jax.ShapeDtypeStruct((P, D), jnp.float32),
    )(gid, xt, wg_all, wu_all, wd_all, w2d)


def _combine_body(og_ref, o_ref):
    o_ref[...] = jnp.sum(og_ref[...], axis=1)


def _combine(og):
    return pl.pallas_call(
        _combine_body,
        grid=(S // TM,),
        in_specs=[pl.BlockSpec((TM, 2 * K, D), lambda i: (i, 0, 0))],
        out_specs=pl.BlockSpec((TM, D), lambda i: (i, 0)),
        out_shape=jax.ShapeDtypeStruct((S, D), jnp.float32),
    )(og)


def kernel(x, W_g, W_gate, W_up, W_down, W_gate_s, W_up_s, W_down_s):
    b, s, d = x.shape
    x_flat = x.reshape(-1, d)

    logits = x_flat @ W_g
    scores = jax.nn.softmax(logits, axis=-1)
    topk_scores, topk_idx = jax.lax.top_k(scores, K)

    e_flat = topk_idx.reshape(-1).astype(jnp.int32)
    w_flat = topk_scores.reshape(-1)
    oh = (e_flat[:, None] == jnp.arange(E, dtype=jnp.int32)[None, :]).astype(
        jnp.int32)
    counts = jnp.sum(oh, axis=0)
    nblk = (counts + TM - 1) // TM
    ends_blk = jnp.cumsum(nblk)
    starts_row = (ends_blk - nblk) * TM
    rank = jnp.take_along_axis(jnp.cumsum(oh, axis=0) - oh,
                               e_flat[:, None], axis=1)[:, 0]
    pos_p = starts_row[e_flat] + rank

    tok_ids = (jnp.arange(S * K, dtype=jnp.int32) // K)
    routed_src = jnp.zeros((P_R,), jnp.int32).at[pos_p].set(tok_ids)
    ar_s = jnp.arange(S, dtype=jnp.int32)
    token_src = jnp.concatenate([routed_src, ar_s, ar_s])

    routed_w = jnp.zeros((P_R,), jnp.float32).at[pos_p].set(w_flat)
    w_pad = jnp.concatenate([routed_w, jnp.ones((N_SHARED * S,), jnp.float32)])

    blk_rows = jnp.arange(NB_R, dtype=jnp.int32) * TM
    gid_r = jnp.clip(
        jnp.searchsorted(ends_blk * TM, blk_rows, side="right"), 0, E - 1
    ).astype(jnp.int32)
    gid = jnp.concatenate([
        gid_r,
        jnp.full((S // TM,), E, jnp.int32),
        jnp.full((S // TM,), E + 1, jnp.int32),
    ])

    pos_tok = pos_p.reshape(S, K)
    pos4 = jnp.concatenate(
        [pos_tok, (P_R + ar_s)[:, None], (P_R + S + ar_s)[:, None]], axis=1
    ).reshape(-1).astype(jnp.int32)

    wg_all = jnp.concatenate(
        [W_gate, W_gate_s.reshape(D, N_SHARED, DE).transpose(1, 0, 2)]
    ).astype(jnp.bfloat16)
    wu_all = jnp.concatenate(
        [W_up, W_up_s.reshape(D, N_SHARED, DE).transpose(1, 0, 2)]
    ).astype(jnp.bfloat16)
    wd_all = jnp.concatenate(
        [W_down, W_down_s.reshape(N_SHARED, DE, D)]
    ).astype(jnp.bfloat16)

    xt = _gather_rows(x_flat, token_src)
    o_routed = _grouped_ffn(gid, xt, wg_all, wu_all, wd_all, w_pad[:, None])
    og = _gather_rows(o_routed, pos4)
    out = _combine(og.reshape(S, 2 * K, D))
    return out.reshape(b, s, d)
